# baseline (speedup 1.0000x reference)
import functools

import jax
import jax.numpy as jnp
from jax import lax
from jax.experimental import pallas as pl
from jax.experimental.pallas import tpu as pltpu
from jax.experimental.pallas import tpu_sc as plsc

_NC = 2
_NS = 16
_NW = _NC * _NS
_CHUNK = 256
_GSUB = 128
_DSTEP = 16


def _sc_rowgather_body(mus_hbm, sig_hbm, z_hbm, muz_hbm, sigz_hbm,
                       z_v, buf_v, gs0, gs1, ss0, ss1):
    N, D = mus_hbm.shape
    B = z_hbm.shape[0]
    b_per_w = B // _NW
    n_g = b_per_w // _CHUNK
    n_sub = _CHUNK // _GSUB

    wid = lax.axis_index("s") * _NC + lax.axis_index("c")
    base = wid * b_per_w

    for r in range(n_g * n_sub):
        pltpu.sync_copy(z_hbm.at[pl.ds(base + r * _GSUB, _GSUB)], z_v.at[r])

    gsems = (gs0, gs1)
    ssems = (ss0, ss1)
    items = [(src, dst, g)
             for src, dst in ((mus_hbm, muz_hbm), (sig_hbm, sigz_hbm))
             for g in range(n_g)]
    n = len(items)
    pend_g = [None, None]
    store_pend = [False, False]

    def issue_gather(i, b):
        src, _, g = items[i]
        pend_g[b] = [
            pltpu.async_copy(src.at[z_v.at[g * n_sub + h]],
                             buf_v.at[b, pl.ds(h * _GSUB, _GSUB)],
                             gsems[b])
            for h in range(n_sub)]

    def drain_stores(b):
        pltpu.make_async_copy(
            mus_hbm.at[pl.ds(0, _CHUNK)], buf_v.at[b], ssems[b]).wait()

    issue_gather(0, 0)
    for i in range(n):
        b = i % 2
        if i + 1 < n:
            b2 = (i + 1) % 2
            if store_pend[b2]:
                drain_stores(b2)
                store_pend[b2] = False
            issue_gather(i + 1, b2)
        for p in pend_g[b]:
            p.wait()
        _, dst, g = items[i]
        off = base + g * _CHUNK

        @pl.loop(0, D, step=_DSTEP)
        def store_cols(d0, b=b, dst=dst, off=off):
            for j in range(_DSTEP):
                d = d0 + j
                pltpu.async_copy(
                    buf_v.at[b, :, d],
                    dst.at[d, pl.ds(off, _CHUNK)],
                    ssems[b])

        store_pend[b] = True
    for b in range(2):
        if store_pend[b]:
            drain_stores(b)


def kernel(mus, sigmas, z):
    D, N = mus.shape
    B = z.shape[0]
    mus_t = mus.T
    sig_t = sigmas.T
    out = jax.ShapeDtypeStruct((D, B), jnp.float32)
    mesh = plsc.VectorSubcoreMesh(core_axis_name="c", subcore_axis_name="s")
    b_per_w = B // _NW
    n_g = b_per_w // _CHUNK
    k = functools.partial(
        pl.kernel,
        out_type=(out, out),
        mesh=mesh,
        scratch_types=[
            pltpu.VMEM((b_per_w // _GSUB, _GSUB), jnp.int32),
            pltpu.VMEM((2, _CHUNK, D), jnp.float32),
            pltpu.SemaphoreType.DMA,
            pltpu.SemaphoreType.DMA,
            pltpu.SemaphoreType.DMA,
            pltpu.SemaphoreType.DMA,
        ],
        compiler_params=pltpu.CompilerParams(needs_layout_passes=False),
    )(_sc_rowgather_body)
    return k(mus_t, sig_t, z)


# speedup vs baseline: 192.2747x; 192.2747x over previous
import functools

import jax
import jax.numpy as jnp
from jax import lax
from jax.experimental import pallas as pl
from jax.experimental.pallas import tpu as pltpu
from jax.experimental.pallas import tpu_sc as plsc

_NC = 2
_NS = 16
_NW = _NC * _NS
_CHUNK = 128
_TBLK = 2048


def _tc_transpose_body(a_ref, b_ref, ao_ref, bo_ref):
    ao_ref[...] = a_ref[...].T
    bo_ref[...] = b_ref[...].T


def _tc_transpose(a, b):
    B, D = a.shape
    out = jax.ShapeDtypeStruct((D, B), jnp.float32)
    return pl.pallas_call(
        _tc_transpose_body,
        grid=(B // _TBLK,),
        in_specs=[pl.BlockSpec((_TBLK, D), lambda i: (i, 0)),
                  pl.BlockSpec((_TBLK, D), lambda i: (i, 0))],
        out_specs=[pl.BlockSpec((D, _TBLK), lambda i: (0, i)),
                   pl.BlockSpec((D, _TBLK), lambda i: (0, i))],
        out_shape=(out, out),
    )(a, b)


def _sc_rowgather_body(mus_hbm, sig_hbm, z_hbm, muz_hbm, sigz_hbm,
                       z_v, rows_v, gs0, gs1, ss0, ss1):
    N, D = mus_hbm.shape
    B = z_hbm.shape[0]
    b_per_w = B // _NW
    n_g = b_per_w // _CHUNK

    wid = lax.axis_index("s") * _NC + lax.axis_index("c")
    base = wid * b_per_w

    for g in range(n_g):
        pltpu.sync_copy(z_hbm.at[pl.ds(base + g * _CHUNK, _CHUNK)], z_v.at[g])

    gsems = (gs0, gs1)
    ssems = (ss0, ss1)
    items = [(src, dst, g)
             for src, dst in ((mus_hbm, muz_hbm), (sig_hbm, sigz_hbm))
             for g in range(n_g)]
    n = len(items)
    pend_g = [None, None]
    pend_s = [None, None]

    def issue_gather(i, b):
        src, _, g = items[i]
        pend_g[b] = pltpu.async_copy(src.at[z_v.at[g]], rows_v.at[b], gsems[b])

    issue_gather(0, 0)
    for i in range(n):
        b = i % 2
        if i + 1 < n:
            b2 = (i + 1) % 2
            if pend_s[b2] is not None:
                pend_s[b2].wait()
                pend_s[b2] = None
            issue_gather(i + 1, b2)
        pend_g[b].wait()
        _, dst, g = items[i]
        pend_s[b] = pltpu.async_copy(
            rows_v.at[b], dst.at[pl.ds(base + g * _CHUNK, _CHUNK)], ssems[b])
    for b in range(2):
        if pend_s[b] is not None:
            pend_s[b].wait()


def kernel(mus, sigmas, z):
    D, N = mus.shape
    B = z.shape[0]
    mus_t = mus.T
    sig_t = sigmas.T
    out_t = jax.ShapeDtypeStruct((B, D), jnp.float32)
    mesh = plsc.VectorSubcoreMesh(core_axis_name="c", subcore_axis_name="s")
    b_per_w = B // _NW
    n_g = b_per_w // _CHUNK
    k = functools.partial(
        pl.kernel,
        out_type=(out_t, out_t),
        mesh=mesh,
        scratch_types=[
            pltpu.VMEM((n_g, _CHUNK), jnp.int32),
            pltpu.VMEM((2, _CHUNK, D), jnp.float32),
            pltpu.SemaphoreType.DMA,
            pltpu.SemaphoreType.DMA,
            pltpu.SemaphoreType.DMA,
            pltpu.SemaphoreType.DMA,
        ],
        compiler_params=pltpu.CompilerParams(needs_layout_passes=False),
    )(_sc_rowgather_body)
    muz_t, sigz_t = k(mus_t, sig_t, z)
    return _tc_transpose(muz_t, sigz_t)
